# baseline (device time: 584298 ns/iter reference)
import functools

import jax
import jax.numpy as jnp
from jax import lax
from jax.experimental import pallas as pl
from jax.experimental.pallas import tpu as pltpu

N_DEV = 4
BF = jnp.bfloat16


def _cast_body(src_ref, dst_ref):
    dst_ref[...] = src_ref[...].astype(dst_ref.dtype)


def _cast(a, dtype, bm=1024):
    m, n = a.shape
    return pl.pallas_call(
        _cast_body,
        grid=(m // bm,),
        in_specs=[pl.BlockSpec((bm, n), lambda i: (i, 0))],
        out_specs=pl.BlockSpec((bm, n), lambda i: (i, 0)),
        out_shape=jax.ShapeDtypeStruct((m, n), dtype),
    )(a)


def _agw_body(wb_ref, wf_ref, copy_sem, cw_send, cw_recv, ccw_send, ccw_recv):
    my = lax.axis_index("i")
    left = (my - 1) % N_DEV
    right = (my + 1) % N_DEV
    k, n_per = wb_ref.shape
    kh = k // 2

    barrier_sem = pltpu.get_barrier_semaphore()
    for nbr in (left, right):
        pl.semaphore_signal(
            barrier_sem, inc=1,
            device_id=(nbr,), device_id_type=pl.DeviceIdType.MESH,
        )
    pl.semaphore_wait(barrier_sem, 2)

    local = pltpu.make_async_copy(
        wb_ref, wf_ref.at[:, pl.ds(my * n_per, n_per)], copy_sem
    )
    local.start()

    for h in range(N_DEV - 1):
        cw_o = (my - h) % N_DEV
        ccw_o = (my + h) % N_DEV
        cw_src = (
            wb_ref.at[pl.ds(0, kh), :]
            if h == 0
            else wf_ref.at[pl.ds(0, kh), pl.ds(cw_o * n_per, n_per)]
        )
        ccw_src = (
            wb_ref.at[pl.ds(kh, kh), :]
            if h == 0
            else wf_ref.at[pl.ds(kh, kh), pl.ds(ccw_o * n_per, n_per)]
        )
        cw = pltpu.make_async_remote_copy(
            src_ref=cw_src,
            dst_ref=wf_ref.at[pl.ds(0, kh), pl.ds(cw_o * n_per, n_per)],
            send_sem=cw_send.at[h],
            recv_sem=cw_recv.at[h],
            device_id=(right,),
            device_id_type=pl.DeviceIdType.MESH,
        )
        ccw = pltpu.make_async_remote_copy(
            src_ref=ccw_src,
            dst_ref=wf_ref.at[pl.ds(kh, kh), pl.ds(ccw_o * n_per, n_per)],
            send_sem=ccw_send.at[h],
            recv_sem=ccw_recv.at[h],
            device_id=(left,),
            device_id_type=pl.DeviceIdType.MESH,
        )
        cw.start()
        ccw.start()
        cw.wait()
        ccw.wait()

    local.wait()


def _ag_w(w_b):
    k, n_per = w_b.shape
    return pl.pallas_call(
        _agw_body,
        out_shape=jax.ShapeDtypeStruct((k, N_DEV * n_per), w_b.dtype),
        in_specs=[pl.BlockSpec(memory_space=pltpu.MemorySpace.HBM)],
        out_specs=pl.BlockSpec(memory_space=pltpu.MemorySpace.HBM),
        scratch_shapes=[
            pltpu.SemaphoreType.DMA,
            pltpu.SemaphoreType.DMA((N_DEV - 1,)),
            pltpu.SemaphoreType.DMA((N_DEV - 1,)),
            pltpu.SemaphoreType.DMA((N_DEV - 1,)),
            pltpu.SemaphoreType.DMA((N_DEV - 1,)),
        ],
        compiler_params=pltpu.CompilerParams(collective_id=0),
    )(w_b)


_BM = 1024
_BN = 4096
_BK = 1024


def _mm_body(x_ref, w_ref, y_ref, acc_ref, *, nk):
    @pl.when(pl.program_id(2) == 0)
    def _():
        acc_ref[...] = jnp.zeros_like(acc_ref)

    acc_ref[...] += jnp.dot(
        x_ref[...], w_ref[...], preferred_element_type=jnp.float32
    )

    @pl.when(pl.program_id(2) == nk - 1)
    def _():
        y_ref[...] = jnp.maximum(acc_ref[...], 0.0).astype(y_ref.dtype)


def _gemm_relu(x_shard, w_full):
    m, k = x_shard.shape
    _, n = w_full.shape
    nk = k // _BK
    return pl.pallas_call(
        functools.partial(_mm_body, nk=nk),
        grid=(m // _BM, n // _BN, nk),
        in_specs=[
            pl.BlockSpec((_BM, _BK), lambda i, j, q: (i, q)),
            pl.BlockSpec((_BK, _BN), lambda i, j, q: (q, j)),
        ],
        out_specs=pl.BlockSpec((_BM, _BN), lambda i, j, q: (i, j)),
        out_shape=jax.ShapeDtypeStruct((m, n), BF),
        scratch_shapes=[pltpu.VMEM((_BM, _BN), jnp.float32)],
        compiler_params=pltpu.CompilerParams(
            dimension_semantics=("parallel", "parallel", "arbitrary"),
            vmem_limit_bytes=100 * 1024 * 1024,
        ),
    )(x_shard, w_full)


def _a2a_body(y_ref, outb_ref, copy_sem, send_sems, recv_sems):
    my = lax.axis_index("i")
    m_loc, n_full = y_ref.shape
    n_per = n_full // N_DEV

    barrier_sem = pltpu.get_barrier_semaphore()
    for d in (1, 2, 3):
        pl.semaphore_signal(
            barrier_sem, inc=1,
            device_id=((my + d) % N_DEV,),
            device_id_type=pl.DeviceIdType.MESH,
        )
    pl.semaphore_wait(barrier_sem, 3)

    local = pltpu.make_async_copy(
        y_ref.at[:, pl.ds(my * n_per, n_per)],
        outb_ref.at[pl.ds(my * m_loc, m_loc), :],
        copy_sem,
    )
    local.start()

    sends = []
    for d in (1, 2, 3):
        tgt = (my + d) % N_DEV
        r = pltpu.make_async_remote_copy(
            src_ref=y_ref.at[:, pl.ds(tgt * n_per, n_per)],
            dst_ref=outb_ref.at[pl.ds(my * m_loc, m_loc), :],
            send_sem=send_sems.at[d - 1],
            recv_sem=recv_sems.at[d - 1],
            device_id=(tgt,),
            device_id_type=pl.DeviceIdType.MESH,
        )
        r.start()
        sends.append(r)
    for r in sends:
        r.wait_send()

    for d in (1, 2, 3):
        src_dev = (my - d) % N_DEV
        recv = pltpu.make_async_remote_copy(
            src_ref=y_ref.at[:, pl.ds(0, n_per)],
            dst_ref=outb_ref.at[pl.ds(src_dev * m_loc, m_loc), :],
            send_sem=send_sems.at[d - 1],
            recv_sem=recv_sems.at[d - 1],
            device_id=(src_dev,),
            device_id_type=pl.DeviceIdType.MESH,
        )
        recv.wait_recv()

    local.wait()


def _a2a(y_b):
    m_loc, n_full = y_b.shape
    n_per = n_full // N_DEV
    return pl.pallas_call(
        _a2a_body,
        out_shape=jax.ShapeDtypeStruct((N_DEV * m_loc, n_per), y_b.dtype),
        in_specs=[pl.BlockSpec(memory_space=pltpu.MemorySpace.HBM)],
        out_specs=pl.BlockSpec(memory_space=pltpu.MemorySpace.HBM),
        scratch_shapes=[
            pltpu.SemaphoreType.DMA,
            pltpu.SemaphoreType.DMA((N_DEV - 1,)),
            pltpu.SemaphoreType.DMA((N_DEV - 1,)),
        ],
        compiler_params=pltpu.CompilerParams(collective_id=1),
    )(y_b)


def kernel(x, w_mat):
    w_b = _cast(w_mat, BF)
    w_full = _ag_w(w_b)
    x_b = _cast(x, BF, bm=256)
    y_b = _gemm_relu(x_b, w_full)
    out_b = _a2a(y_b)
    return _cast(out_b, jnp.float32)


# device time: 484609 ns/iter; 1.2057x vs baseline; 1.2057x over previous
import functools

import jax
import jax.numpy as jnp
from jax import lax
from jax.experimental import pallas as pl
from jax.experimental.pallas import tpu as pltpu

N_DEV = 4
BF = jnp.bfloat16


def _cast_body(src_ref, dst_ref):
    dst_ref[...] = src_ref[...].astype(dst_ref.dtype)


def _cast(a, dtype, bm=1024):
    m, n = a.shape
    return pl.pallas_call(
        _cast_body,
        grid=(m // bm,),
        in_specs=[pl.BlockSpec((bm, n), lambda i: (i, 0))],
        out_specs=pl.BlockSpec((bm, n), lambda i: (i, 0)),
        out_shape=jax.ShapeDtypeStruct((m, n), dtype),
    )(a)


def _agw_body(wb_ref, wf_ref, copy_sem, cw_send, cw_recv, ccw_send, ccw_recv):
    my = lax.axis_index("i")
    left = (my - 1) % N_DEV
    right = (my + 1) % N_DEV
    k, n_per = wb_ref.shape
    kh = k // 2

    barrier_sem = pltpu.get_barrier_semaphore()
    for nbr in (left, right):
        pl.semaphore_signal(
            barrier_sem, inc=1,
            device_id=(nbr,), device_id_type=pl.DeviceIdType.MESH,
        )
    pl.semaphore_wait(barrier_sem, 2)

    local = pltpu.make_async_copy(
        wb_ref, wf_ref.at[:, pl.ds(3 * n_per, n_per)], copy_sem
    )
    local.start()

    for h in range(N_DEV - 1):
        cw_src = (
            wb_ref.at[pl.ds(0, kh), :]
            if h == 0
            else wf_ref.at[pl.ds(0, kh), pl.ds(((-h - 1) % N_DEV) * n_per, n_per)]
        )
        cw_dst_p = (-h - 2) % N_DEV
        ccw_src = (
            wb_ref.at[pl.ds(kh, kh), :]
            if h == 0
            else wf_ref.at[pl.ds(kh, kh), pl.ds((h - 1) * n_per, n_per)]
        )
        ccw_dst_p = h
        cw = pltpu.make_async_remote_copy(
            src_ref=cw_src,
            dst_ref=wf_ref.at[pl.ds(0, kh), pl.ds(cw_dst_p * n_per, n_per)],
            send_sem=cw_send.at[h],
            recv_sem=cw_recv.at[h],
            device_id=(right,),
            device_id_type=pl.DeviceIdType.MESH,
        )
        ccw = pltpu.make_async_remote_copy(
            src_ref=ccw_src,
            dst_ref=wf_ref.at[pl.ds(kh, kh), pl.ds(ccw_dst_p * n_per, n_per)],
            send_sem=ccw_send.at[h],
            recv_sem=ccw_recv.at[h],
            device_id=(left,),
            device_id_type=pl.DeviceIdType.MESH,
        )
        cw.start()
        ccw.start()
        cw.wait()
        ccw.wait()

    local.wait()


def _ag_w(w_b):
    k, n_per = w_b.shape
    return pl.pallas_call(
        _agw_body,
        out_shape=jax.ShapeDtypeStruct((k, N_DEV * n_per), w_b.dtype),
        in_specs=[pl.BlockSpec(memory_space=pltpu.MemorySpace.HBM)],
        out_specs=pl.BlockSpec(memory_space=pltpu.MemorySpace.HBM),
        scratch_shapes=[
            pltpu.SemaphoreType.DMA,
            pltpu.SemaphoreType.DMA((N_DEV - 1,)),
            pltpu.SemaphoreType.DMA((N_DEV - 1,)),
            pltpu.SemaphoreType.DMA((N_DEV - 1,)),
            pltpu.SemaphoreType.DMA((N_DEV - 1,)),
        ],
        compiler_params=pltpu.CompilerParams(collective_id=0),
    )(w_b)


_BK = 1024


def _mm_a2a_body(
    x_ref, w_ref, outb_ref, stage_ref, acc_ref, copy_sem, send_sems, recv_sems,
    *, nk,
):
    j = pl.program_id(0)
    k = pl.program_id(1)
    my = lax.axis_index("i")
    m_loc = x_ref.shape[0]

    @pl.when((j == 0) & (k == 0))
    def _():
        barrier_sem = pltpu.get_barrier_semaphore()
        for d in (1, 2, 3):
            pl.semaphore_signal(
                barrier_sem, inc=1,
                device_id=((my + d) % N_DEV,),
                device_id_type=pl.DeviceIdType.MESH,
            )
        pl.semaphore_wait(barrier_sem, 3)

    @pl.when(k == 0)
    def _():
        acc_ref[...] = jnp.zeros_like(acc_ref)

    acc_ref[...] += jnp.dot(
        x_ref[...].astype(BF), w_ref[...], preferred_element_type=jnp.float32
    )

    @pl.when(k == nk - 1)
    def _():
        stage_ref[j] = jnp.maximum(acc_ref[...], 0.0).astype(BF)
        dst = outb_ref.at[pl.ds(my * m_loc, m_loc), :]

        @pl.when(j == N_DEV - 1)
        def _():
            pltpu.make_async_copy(stage_ref.at[j], dst, copy_sem).start()

        @pl.when(j != N_DEV - 1)
        def _():
            pltpu.make_async_remote_copy(
                src_ref=stage_ref.at[j],
                dst_ref=dst,
                send_sem=send_sems.at[j],
                recv_sem=recv_sems.at[j],
                device_id=((my + 1 + j) % N_DEV,),
                device_id_type=pl.DeviceIdType.MESH,
            ).start()

    @pl.when((j == N_DEV - 1) & (k == nk - 1))
    def _():
        dst_own = outb_ref.at[pl.ds(my * m_loc, m_loc), :]
        pltpu.make_async_copy(stage_ref.at[N_DEV - 1], dst_own, copy_sem).wait()
        for jj in range(N_DEV - 1):
            pltpu.make_async_remote_copy(
                src_ref=stage_ref.at[jj],
                dst_ref=dst_own,
                send_sem=send_sems.at[jj],
                recv_sem=recv_sems.at[jj],
                device_id=((my + 1 + jj) % N_DEV,),
                device_id_type=pl.DeviceIdType.MESH,
            ).wait_send()
        for jj in range(N_DEV - 1):
            src_dev = (my - 1 - jj) % N_DEV
            pltpu.make_async_remote_copy(
                src_ref=stage_ref.at[jj],
                dst_ref=outb_ref.at[pl.ds(src_dev * m_loc, m_loc), :],
                send_sem=send_sems.at[jj],
                recv_sem=recv_sems.at[jj],
                device_id=(src_dev,),
                device_id_type=pl.DeviceIdType.MESH,
            ).wait_recv()


def _gemm_relu_a2a(x_shard, w_full):
    m_loc, kdim = x_shard.shape
    _, n = w_full.shape
    n_per = n // N_DEV
    nk = kdim // _BK
    return pl.pallas_call(
        functools.partial(_mm_a2a_body, nk=nk),
        grid=(N_DEV, nk),
        in_specs=[
            pl.BlockSpec((m_loc, _BK), lambda j, q: (0, q)),
            pl.BlockSpec((_BK, n_per), lambda j, q: (q, j)),
        ],
        out_specs=pl.BlockSpec(memory_space=pltpu.MemorySpace.HBM),
        out_shape=jax.ShapeDtypeStruct((N_DEV * m_loc, n_per), BF),
        scratch_shapes=[
            pltpu.VMEM((N_DEV, m_loc, n_per), BF),
            pltpu.VMEM((m_loc, n_per), jnp.float32),
            pltpu.SemaphoreType.DMA,
            pltpu.SemaphoreType.DMA((N_DEV - 1,)),
            pltpu.SemaphoreType.DMA((N_DEV - 1,)),
        ],
        compiler_params=pltpu.CompilerParams(
            dimension_semantics=("arbitrary", "arbitrary"),
            vmem_limit_bytes=100 * 1024 * 1024,
            collective_id=1,
        ),
    )(x_shard, w_full)


def kernel(x, w_mat):
    w_b = _cast(w_mat, BF)
    w_full = _ag_w(w_b)
    out_b = _gemm_relu_a2a(x, w_full)
    return _cast(out_b, jnp.float32)
